# Initial kernel scaffold; baseline (speedup 1.0000x reference)
import jax
import jax.numpy as jnp
from jax.experimental import pallas as pl


def kernel(behavior_emb, target_emb, seq_len, W1, b1, W2, b2):
    raise NotImplementedError("write your pallas kernel here")


# speedup vs baseline: 1.8460x; 1.8460x over previous
import jax
import jax.numpy as jnp
from jax.experimental import pallas as pl
from jax.experimental.pallas import tpu as pltpu

_B, _L, _E, _H = 4096, 200, 64, 128
_BB = 64


def _attn_block(x_ref, t_ref, seq_ref, W1_ref, b1_ref, W2_ref, o_ref):
    x3 = x_ref[...]
    t3 = t_ref[...]
    W1 = W1_ref[...]
    Wa = W1[0:_E] + W1[3 * _E:4 * _E]
    Wc = W1[2 * _E:3 * _E]
    Wt = W1[_E:2 * _E] - W1[3 * _E:4 * _E]
    b1 = b1_ref[...]
    W2 = W2_ref[...]
    seq = seq_ref[...].reshape(_BB, 1, 1)

    x2 = x3.reshape(_BB * _L, _E)
    xp2 = (x3 * t3).reshape(_BB * _L, _E)
    c = jnp.dot(t3.reshape(_BB, _E), Wt,
                preferred_element_type=jnp.float32) + b1
    h2 = (jnp.dot(x2, Wa, preferred_element_type=jnp.float32)
          + jnp.dot(xp2, Wc, preferred_element_type=jnp.float32))
    h3 = jnp.maximum(h2.reshape(_BB, _L, _H) + c.reshape(_BB, 1, _H), 0.0)
    s2 = jnp.dot(h3.reshape(_BB * _L, _H), W2,
                 preferred_element_type=jnp.float32)
    s3 = s2.reshape(_BB, _L, 1)

    l_idx = jax.lax.broadcasted_iota(jnp.int32, (1, _L, 1), 1)
    mask3 = l_idx < seq
    e3 = jnp.where(mask3, jnp.exp(s3), 0.0) + jnp.where(seq == 0, 1.0, 0.0)
    denom = jnp.sum(e3, axis=1, keepdims=True)
    att3 = e3 / denom
    o_ref[...] = jnp.sum(att3 * x3, axis=1)


def kernel(behavior_emb, target_emb, seq_len, W1, b1, W2, b2):
    del b2
    nb = _B // _BB
    seq3 = seq_len.astype(jnp.int32).reshape(nb, _BB, 1)
    b1r = b1.reshape(1, _H)
    return pl.pallas_call(
        _attn_block,
        out_shape=jax.ShapeDtypeStruct((_B, _E), jnp.float32),
        grid=(nb,),
        in_specs=[
            pl.BlockSpec((_BB, _L, _E), lambda i: (i, 0, 0)),
            pl.BlockSpec((_BB, 1, _E), lambda i: (i, 0, 0)),
            pl.BlockSpec((1, _BB, 1), lambda i: (i, 0, 0)),
            pl.BlockSpec((4 * _E, _H), lambda i: (0, 0)),
            pl.BlockSpec((1, _H), lambda i: (0, 0)),
            pl.BlockSpec((_H, 1), lambda i: (0, 0)),
        ],
        out_specs=pl.BlockSpec((_BB, _E), lambda i: (i, 0)),
        compiler_params=pltpu.CompilerParams(
            dimension_semantics=("parallel",),
            vmem_limit_bytes=64 * 1024 * 1024,
        ),
    )(behavior_emb, target_emb, seq3, W1, b1r, W2)
